# baseline (speedup 1.0000x reference)
import functools

import jax
import jax.numpy as jnp
from jax import lax
from jax.experimental import pallas as pl
from jax.experimental.pallas import tpu as pltpu

HIDDEN = 2048
N_EXP = 8
BLKM = 1024
NBUF = 4
NOUT = 2


def _tc_kernel(x_hbm, w_ref, o_hbm, xbufs, obufs, insem, outsem):
    nblk = x_hbm.shape[0] // BLKM

    def in_copy(b, buf):
        return pltpu.make_async_copy(
            x_hbm.at[pl.ds(b * BLKM, BLKM)], xbufs.at[buf], insem.at[buf])

    def out_copy(b, obuf):
        return pltpu.make_async_copy(
            obufs.at[obuf], o_hbm.at[pl.ds(b * BLKM, BLKM)], outsem.at[obuf])

    for i in range(NBUF):
        in_copy(i, i).start()

    @pl.loop(0, nblk)
    def _(b):
        buf = lax.rem(b, NBUF)
        obuf = lax.rem(b, NOUT)
        in_copy(b, buf).wait()

        @pl.when(b >= NOUT)
        def _():
            out_copy(b - NOUT, obuf).wait()

        obufs[obuf] = jax.lax.dot_general(
            xbufs[buf], w_ref[...],
            dimension_numbers=(((1,), (1,)), ((), ())),
            preferred_element_type=jnp.float32,
        )
        out_copy(b, obuf).start()

        @pl.when(b + NBUF < nblk)
        def _():
            in_copy(b + NBUF, buf).start()

    for i in range(NOUT):
        out_copy(nblk - NOUT + i, lax.rem(nblk - NOUT + i, NOUT)).wait()


def kernel(x, weight):
    xf = x.reshape(-1, HIDDEN)
    rows = xf.shape[0]
    out = pl.pallas_call(
        _tc_kernel,
        in_specs=[
            pl.BlockSpec(memory_space=pl.MemorySpace.ANY),
            pl.BlockSpec((N_EXP, HIDDEN), lambda: (0, 0)),
        ],
        out_specs=pl.BlockSpec(memory_space=pl.MemorySpace.ANY),
        out_shape=jax.ShapeDtypeStruct((rows, N_EXP), jnp.float32),
        scratch_shapes=[
            pltpu.VMEM((NBUF, BLKM, HIDDEN), jnp.float32),
            pltpu.VMEM((NOUT, BLKM, N_EXP), jnp.float32),
            pltpu.SemaphoreType.DMA((NBUF,)),
            pltpu.SemaphoreType.DMA((NOUT,)),
        ],
    )(xf, weight)
    return out
